# baseline (speedup 1.0000x reference)
import functools

import jax
import jax.numpy as jnp
import numpy as np
from jax import lax
from jax.experimental import pallas as pl
from jax.experimental.pallas import tpu as pltpu
from jax.experimental.pallas import tpu_sc as plsc

RADIUS = 0.3
NSAMPLE = 16
NPOINT = 512
EPS = 1e-5

B = 4
N = 16384
C = 256
SUB = 8
NL = N // SUB
D_ROW = 128
C_TILE = 8
M_TILE = 64


def _rmax(x):
    return jnp.max(jnp.max(x, axis=1, keepdims=True), axis=0, keepdims=True)


def _rmin(x):
    return jnp.min(jnp.min(x, axis=1, keepdims=True), axis=0, keepdims=True)


def _rsum(x):
    return jnp.sum(jnp.sum(x, axis=1, keepdims=True), axis=0, keepdims=True)


def _fps_body(seed_ref, rows_ref, inds_ref):
    sx = seed_ref[0, 0]
    sy = seed_ref[0, 1]
    sz = seed_ref[0, 2]
    lin = (lax.broadcasted_iota(jnp.int32, (SUB, NL), 0) * NL
           + lax.broadcasted_iota(jnp.int32, (SUB, NL), 1))
    lin512 = (lax.broadcasted_iota(jnp.int32, (SUB, 64), 0) * 64
              + lax.broadcasted_iota(jnp.int32, (SUB, 64), 1))

    def body(i, state):
        dist, f, inds = state
        inds = jnp.where(lin512 == i, f, inds)
        row = rows_ref[0, pl.ds(f[0, 0], 1), :]
        cx = row[:, 0:1]
        cy = row[:, 1:2]
        cz = row[:, 2:3]
        dx = sx - cx
        dy = sy - cy
        dz = sz - cz
        d = (dx * dx + dy * dy) + dz * dz
        dist = jnp.minimum(dist, d)
        m = _rmax(dist)
        cand = jnp.where(dist == m, lin, N)
        f_next = _rmin(cand)
        return dist, f_next, inds

    init = (jnp.full((SUB, NL), 1e10, jnp.float32),
            jnp.zeros((1, 1), jnp.int32),
            jnp.zeros((SUB, 64), jnp.int32))
    _, _, inds = lax.fori_loop(0, NPOINT, body, init)
    inds_ref[0] = inds


def _run_fps(seed_t4, seed_xyz):
    return pl.pallas_call(
        _fps_body,
        grid=(B,),
        in_specs=[
            pl.BlockSpec((1, 3, SUB, NL), lambda b: (b, 0, 0, 0)),
            pl.BlockSpec((1, N, 3), lambda b: (b, 0, 0)),
        ],
        out_specs=pl.BlockSpec((1, SUB, 64), lambda b: (b, 0, 0)),
        out_shape=jax.ShapeDtypeStruct((B, SUB, 64), jnp.int32),
    )(seed_t4, seed_xyz)


def _ballq_body(xyz_ref, rows_ref, inds_ref, idx_ref, nxyz_ref, ctr_scr):
    b = pl.program_id(0)
    x = xyz_ref[0, 0:1, :]
    y = xyz_ref[0, 1:2, :]
    z = xyz_ref[0, 2:3, :]
    t = pl.program_id(1)
    for j in range(C_TILE):
        s = inds_ref[0, t, j]
        ctr_scr[pl.ds(j, 1), :] = rows_ref[0, pl.ds(s, 1), :]
    c = ctr_scr[...]
    nxyz_ref[0] = c
    cx = c[:, 0:1]
    cy = c[:, 1:2]
    cz = c[:, 2:3]
    dx = cx - x
    dy = cy - y
    dz = cz - z
    d2 = (dx * dx + dy * dy) + dz * dz
    lin = lax.broadcasted_iota(jnp.int32, (C_TILE, N), 1)
    scores = jnp.where(d2 < np.float32(RADIUS * RADIUS), lin, N)
    iota16 = lax.broadcasted_iota(jnp.int32, (1, NSAMPLE), 1)
    idxmat = jnp.zeros((C_TILE, NSAMPLE), jnp.int32)
    first = jnp.full((C_TILE, 1), N, jnp.int32)
    for k in range(NSAMPLE):
        m = jnp.min(scores, axis=1, keepdims=True)
        if k == 0:
            first = m
        slotval = jnp.where(m < N, m, first)
        scores = jnp.where(scores == m, N, scores)
        idxmat = jnp.where(iota16 == k, slotval, idxmat)
    idx_ref[0] = idxmat + b * N


def _run_ballq(xyz_t3, xyz, inds3):
    return pl.pallas_call(
        _ballq_body,
        grid=(B, NPOINT // C_TILE),
        in_specs=[
            pl.BlockSpec((1, 3, N), lambda b, t: (b, 0, 0)),
            pl.BlockSpec((1, N, 3), lambda b, t: (b, 0, 0)),
            pl.BlockSpec((1, NPOINT // C_TILE, C_TILE), lambda b, t: (b, 0, 0),
                         memory_space=pltpu.SMEM),
        ],
        out_specs=[
            pl.BlockSpec((1, C_TILE, NSAMPLE), lambda b, t: (b, t, 0)),
            pl.BlockSpec((1, C_TILE, 3), lambda b, t: (b, t, 0)),
        ],
        out_shape=[
            jax.ShapeDtypeStruct((B, NPOINT, NSAMPLE), jnp.int32),
            jax.ShapeDtypeStruct((B, NPOINT, 3), jnp.float32),
        ],
        scratch_shapes=[pltpu.VMEM((C_TILE, 3), jnp.float32)],
    )(xyz_t3, xyz, inds3)


def _table_body(f_ref, xyz_ref, wt_ref, wg_ref, out_ref):
    f = f_ref[0]
    wt = wt_ref[...]
    out128 = lax.dot_general(f, wt, (((0,), (0,)), ((), ())),
                             preferred_element_type=jnp.float32)
    xyzb = xyz_ref[0]
    wg = wg_ref[...]
    xyzproj = (xyzb[:, 0:1] * wg[0:1, :] + xyzb[:, 1:2] * wg[1:2, :]
               + xyzb[:, 2:3] * wg[2:3, :])
    out_ref[0] = out128 + xyzproj


def _run_table(features, xyz, w1f_t, wg):
    n_tile = 512
    return pl.pallas_call(
        _table_body,
        grid=(B, N // n_tile),
        in_specs=[
            pl.BlockSpec((1, C, n_tile), lambda b, t: (b, 0, t)),
            pl.BlockSpec((1, n_tile, 3), lambda b, t: (b, t, 0)),
            pl.BlockSpec((C, 128), lambda b, t: (0, 0)),
            pl.BlockSpec((3, 128), lambda b, t: (0, 0)),
        ],
        out_specs=pl.BlockSpec((1, n_tile, D_ROW), lambda b, t: (b, t, 0)),
        out_shape=jax.ShapeDtypeStruct((B, N, D_ROW), jnp.float32),
    )(features, xyz, w1f_t, wg)


_SC_CHUNK = 128


def _sc_gather(table, idx):
    tot = idx.shape[0]
    info = plsc.get_sparse_core_info()
    nw = info.num_cores * info.num_subcores
    per_w = tot // nw
    n_chunk = per_w // _SC_CHUNK
    mesh = plsc.VectorSubcoreMesh(core_axis_name="c", subcore_axis_name="s")

    @functools.partial(
        pl.kernel,
        mesh=mesh,
        out_type=jax.ShapeDtypeStruct((tot, D_ROW), jnp.float32),
        scratch_types=[
            pltpu.VMEM((_SC_CHUNK,), jnp.int32),
            pltpu.VMEM((_SC_CHUNK, D_ROW), jnp.float32),
            pltpu.SemaphoreType.DMA,
        ],
    )
    def k(tab_hbm, idx_hbm, out_hbm, idx_v, rows_v, sem):
        wid = lax.axis_index("s") * info.num_cores + lax.axis_index("c")
        base = wid * per_w

        def body(ci, carry):
            off = base + ci * _SC_CHUNK
            pltpu.sync_copy(idx_hbm.at[pl.ds(off, _SC_CHUNK)], idx_v)
            pltpu.async_copy(tab_hbm.at[idx_v], rows_v, sem).wait()
            pltpu.sync_copy(rows_v, out_hbm.at[pl.ds(off, _SC_CHUNK)])
            return carry

        lax.fori_loop(0, n_chunk, body, 0)

    return k(table, idx)


def _mlp_body(g_ref, ctr_ref, wg_ref, w2_ref, w3_ref,
              o1_ref, o2_ref, o3_ref, out_ref):
    h1p = g_ref[0]
    c = ctr_ref[0]
    wg = wg_ref[...]
    cc = (c[:, 0:1] * wg[0:1, :] + c[:, 1:2] * wg[1:2, :]
          + c[:, 2:3] * wg[2:3, :])
    ccrep = jnp.reshape(
        jnp.broadcast_to(cc[:, None, :], (M_TILE, NSAMPLE, 128)),
        (M_TILE * NSAMPLE, 128))
    h1 = jnp.maximum(h1p - ccrep + o1_ref[...], 0.0)
    h2 = jnp.maximum(
        lax.dot_general(h1, w2_ref[...], (((1,), (0,)), ((), ())),
                        preferred_element_type=jnp.float32) + o2_ref[...],
        0.0)
    h3 = jnp.maximum(
        lax.dot_general(h2, w3_ref[...], (((1,), (0,)), ((), ())),
                        preferred_element_type=jnp.float32) + o3_ref[...],
        0.0)
    out_ref[0] = jnp.max(jnp.reshape(h3, (M_TILE, NSAMPLE, 128)), axis=1)


def _run_mlp(grows, centers, wg, w2t, w3t, o1, o2, o3):
    rows_tile = M_TILE * NSAMPLE
    return pl.pallas_call(
        _mlp_body,
        grid=(B, NPOINT // M_TILE),
        in_specs=[
            pl.BlockSpec((1, rows_tile, D_ROW), lambda b, t: (b, t, 0)),
            pl.BlockSpec((1, M_TILE, 3), lambda b, t: (b, t, 0)),
            pl.BlockSpec((3, 128), lambda b, t: (0, 0)),
            pl.BlockSpec((128, 128), lambda b, t: (0, 0)),
            pl.BlockSpec((128, 128), lambda b, t: (0, 0)),
            pl.BlockSpec((1, 128), lambda b, t: (0, 0)),
            pl.BlockSpec((1, 128), lambda b, t: (0, 0)),
            pl.BlockSpec((1, 128), lambda b, t: (0, 0)),
        ],
        out_specs=pl.BlockSpec((1, M_TILE, 128), lambda b, t: (b, t, 0)),
        out_shape=jax.ShapeDtypeStruct((B, NPOINT, 128), jnp.float32),
    )(grows, centers, wg, w2t, w3t, o1, o2, o3)


def kernel(xyz, features, seed_xyz, W1, g1, b1, m1, v1,
           W2, g2, b2, m2, v2, W3, g3, b3, m3, v3):
    xyz_t3 = jnp.transpose(xyz, (0, 2, 1))
    seed_t4 = jnp.reshape(jnp.transpose(seed_xyz, (0, 2, 1)), (B, 3, SUB, NL))

    s1 = g1 / jnp.sqrt(v1 + EPS)
    s2 = g2 / jnp.sqrt(v2 + EPS)
    s3 = g3 / jnp.sqrt(v3 + EPS)
    o1 = (b1 - m1 * s1)[None, :]
    o2 = (b2 - m2 * s2)[None, :]
    o3 = (b3 - m3 * s3)[None, :]
    w1f_t = jnp.transpose(W1[:, 3:] * s1[:, None])
    wg = jnp.transpose(W1[:, :3] * (s1 / RADIUS)[:, None])
    w2t = jnp.transpose(W2 * s2[:, None])
    w3t = jnp.transpose(W3 * s3[:, None])

    inds8 = _run_fps(seed_t4, seed_xyz)
    sample_inds = jnp.reshape(inds8, (B, NPOINT))
    inds3 = jnp.reshape(sample_inds, (B, NPOINT // C_TILE, C_TILE))

    idx, new_xyz = _run_ballq(xyz_t3, xyz, inds3)
    table = _run_table(features, xyz, w1f_t, wg)

    grows = _sc_gather(jnp.reshape(table, (B * N, D_ROW)),
                       jnp.reshape(idx, (B * NPOINT * NSAMPLE,)))
    out = _run_mlp(jnp.reshape(grows, (B, NPOINT * NSAMPLE, D_ROW)),
                   new_xyz, wg, w2t, w3t, o1, o2, o3)
    new_features = jnp.transpose(out, (0, 2, 1))
    return (new_xyz, new_features, sample_inds)


# speedup vs baseline: 1.8944x; 1.8944x over previous
import functools

import jax
import jax.numpy as jnp
import numpy as np
from jax import lax
from jax.experimental import pallas as pl
from jax.experimental.pallas import tpu as pltpu
from jax.experimental.pallas import tpu_sc as plsc

RADIUS = 0.3
NSAMPLE = 16
NPOINT = 512
EPS = 1e-5

B = 4
N = 16384
C = 256
SUB = 8
NL = N // SUB
D_ROW = 128
C_TILE = 32
M_TILE = 64


def _rmax(x):
    return jnp.max(jnp.max(x, axis=1, keepdims=True), axis=0, keepdims=True)


def _rmin(x):
    return jnp.min(jnp.min(x, axis=1, keepdims=True), axis=0, keepdims=True)


def _rsum(x):
    return jnp.sum(jnp.sum(x, axis=1, keepdims=True), axis=0, keepdims=True)


def _fps_body(seed_ref, rows_ref, inds_ref):
    sx = [seed_ref[b, 0] for b in range(B)]
    sy = [seed_ref[b, 1] for b in range(B)]
    sz = [seed_ref[b, 2] for b in range(B)]
    lin = (lax.broadcasted_iota(jnp.int32, (SUB, NL), 0) * NL
           + lax.broadcasted_iota(jnp.int32, (SUB, NL), 1))
    lin512 = (lax.broadcasted_iota(jnp.int32, (SUB, 64), 0) * 64
              + lax.broadcasted_iota(jnp.int32, (SUB, 64), 1))

    def body(i, state):
        dists, fs, inds = state
        onehot = lin512 == i
        new_dists, new_fs, new_inds = [], [], []
        for b in range(B):
            f = fs[b]
            ind_b = jnp.where(onehot, f, inds[b])
            row = rows_ref[b, pl.ds(f[0, 0], 1), :]
            dx = sx[b] - row[:, 0:1]
            dy = sy[b] - row[:, 1:2]
            dz = sz[b] - row[:, 2:3]
            d = (dx * dx + dy * dy) + dz * dz
            dist = jnp.minimum(dists[b], d)
            m = _rmax(dist)
            cand = jnp.where(dist == m, lin, N)
            new_dists.append(dist)
            new_fs.append(_rmin(cand))
            new_inds.append(ind_b)
        return tuple(new_dists), tuple(new_fs), tuple(new_inds)

    init = (tuple(jnp.full((SUB, NL), 1e10, jnp.float32) for _ in range(B)),
            tuple(jnp.zeros((1, 1), jnp.int32) for _ in range(B)),
            tuple(jnp.zeros((SUB, 64), jnp.int32) for _ in range(B)))
    _, _, inds = lax.fori_loop(0, NPOINT, body, init)
    for b in range(B):
        inds_ref[b] = inds[b]


def _run_fps(seed_t4, seed_xyz):
    return pl.pallas_call(
        _fps_body,
        in_specs=[
            pl.BlockSpec((B, 3, SUB, NL), lambda: (0, 0, 0, 0)),
            pl.BlockSpec((B, N, 3), lambda: (0, 0, 0)),
        ],
        out_specs=pl.BlockSpec((B, SUB, 64), lambda: (0, 0, 0)),
        out_shape=jax.ShapeDtypeStruct((B, SUB, 64), jnp.int32),
    )(seed_t4, seed_xyz)


def _ballq_body(xyz_ref, rows_ref, inds_ref, idx_ref, nxyz_ref, ctr_scr):
    b = pl.program_id(0)
    x = xyz_ref[0, 0:1, :]
    y = xyz_ref[0, 1:2, :]
    z = xyz_ref[0, 2:3, :]
    t = pl.program_id(1)
    for j in range(C_TILE):
        s = inds_ref[0, t, j]
        ctr_scr[pl.ds(j, 1), :] = rows_ref[0, pl.ds(s, 1), :]
    c = ctr_scr[...]
    nxyz_ref[0] = c
    cx = c[:, 0:1]
    cy = c[:, 1:2]
    cz = c[:, 2:3]
    dx = cx - x
    dy = cy - y
    dz = cz - z
    d2 = (dx * dx + dy * dy) + dz * dz
    lin = lax.broadcasted_iota(jnp.int32, (C_TILE, N), 1)
    scores = jnp.where(d2 < np.float32(RADIUS * RADIUS), lin, N)
    iota16 = lax.broadcasted_iota(jnp.int32, (1, NSAMPLE), 1)
    idxmat = jnp.zeros((C_TILE, NSAMPLE), jnp.int32)
    first = jnp.full((C_TILE, 1), N, jnp.int32)
    for k in range(NSAMPLE):
        m = jnp.min(scores, axis=1, keepdims=True)
        if k == 0:
            first = m
        slotval = jnp.where(m < N, m, first)
        scores = jnp.where(scores == m, N, scores)
        idxmat = jnp.where(iota16 == k, slotval, idxmat)
    idx_ref[0] = idxmat + b * N


def _run_ballq(xyz_t3, xyz, inds3):
    return pl.pallas_call(
        _ballq_body,
        grid=(B, NPOINT // C_TILE),
        in_specs=[
            pl.BlockSpec((1, 3, N), lambda b, t: (b, 0, 0)),
            pl.BlockSpec((1, N, 3), lambda b, t: (b, 0, 0)),
            pl.BlockSpec((1, NPOINT // C_TILE, C_TILE), lambda b, t: (b, 0, 0),
                         memory_space=pltpu.SMEM),
        ],
        out_specs=[
            pl.BlockSpec((1, C_TILE, NSAMPLE), lambda b, t: (b, t, 0)),
            pl.BlockSpec((1, C_TILE, 3), lambda b, t: (b, t, 0)),
        ],
        out_shape=[
            jax.ShapeDtypeStruct((B, NPOINT, NSAMPLE), jnp.int32),
            jax.ShapeDtypeStruct((B, NPOINT, 3), jnp.float32),
        ],
        scratch_shapes=[pltpu.VMEM((C_TILE, 3), jnp.float32)],
    )(xyz_t3, xyz, inds3)


def _table_body(f_ref, xyz_ref, wt_ref, wg_ref, out_ref):
    f = f_ref[0]
    wt = wt_ref[...]
    out128 = lax.dot_general(f, wt, (((0,), (0,)), ((), ())),
                             preferred_element_type=jnp.float32)
    xyzb = xyz_ref[0]
    wg = wg_ref[...]
    xyzproj = (xyzb[:, 0:1] * wg[0:1, :] + xyzb[:, 1:2] * wg[1:2, :]
               + xyzb[:, 2:3] * wg[2:3, :])
    out_ref[0] = out128 + xyzproj


def _run_table(features, xyz, w1f_t, wg):
    n_tile = 512
    return pl.pallas_call(
        _table_body,
        grid=(B, N // n_tile),
        in_specs=[
            pl.BlockSpec((1, C, n_tile), lambda b, t: (b, 0, t)),
            pl.BlockSpec((1, n_tile, 3), lambda b, t: (b, t, 0)),
            pl.BlockSpec((C, 128), lambda b, t: (0, 0)),
            pl.BlockSpec((3, 128), lambda b, t: (0, 0)),
        ],
        out_specs=pl.BlockSpec((1, n_tile, D_ROW), lambda b, t: (b, t, 0)),
        out_shape=jax.ShapeDtypeStruct((B, N, D_ROW), jnp.float32),
    )(features, xyz, w1f_t, wg)


_SC_CHUNK = 128


def _sc_gather(table, idx):
    tot = idx.shape[0]
    info = plsc.get_sparse_core_info()
    nw = info.num_cores * info.num_subcores
    per_w = tot // nw
    n_chunk = per_w // _SC_CHUNK
    mesh = plsc.VectorSubcoreMesh(core_axis_name="c", subcore_axis_name="s")

    @functools.partial(
        pl.kernel,
        mesh=mesh,
        out_type=jax.ShapeDtypeStruct((tot, D_ROW), jnp.float32),
        scratch_types=[
            pltpu.VMEM((_SC_CHUNK,), jnp.int32),
            pltpu.VMEM((_SC_CHUNK, D_ROW), jnp.float32),
            pltpu.SemaphoreType.DMA,
        ],
    )
    def k(tab_hbm, idx_hbm, out_hbm, idx_v, rows_v, sem):
        wid = lax.axis_index("s") * info.num_cores + lax.axis_index("c")
        base = wid * per_w

        def body(ci, carry):
            off = base + ci * _SC_CHUNK
            pltpu.sync_copy(idx_hbm.at[pl.ds(off, _SC_CHUNK)], idx_v)
            pltpu.async_copy(tab_hbm.at[idx_v], rows_v, sem).wait()
            pltpu.sync_copy(rows_v, out_hbm.at[pl.ds(off, _SC_CHUNK)])
            return carry

        lax.fori_loop(0, n_chunk, body, 0)

    return k(table, idx)


def _mlp_body(g_ref, ctr_ref, wg_ref, w2_ref, w3_ref,
              o1_ref, o2_ref, o3_ref, out_ref):
    h1p = g_ref[0]
    c = ctr_ref[0]
    wg = wg_ref[...]
    cc = (c[:, 0:1] * wg[0:1, :] + c[:, 1:2] * wg[1:2, :]
          + c[:, 2:3] * wg[2:3, :])
    ccrep = jnp.reshape(
        jnp.broadcast_to(cc[:, None, :], (M_TILE, NSAMPLE, 128)),
        (M_TILE * NSAMPLE, 128))
    h1 = jnp.maximum(h1p - ccrep + o1_ref[...], 0.0)
    h2 = jnp.maximum(
        lax.dot_general(h1, w2_ref[...], (((1,), (0,)), ((), ())),
                        preferred_element_type=jnp.float32) + o2_ref[...],
        0.0)
    h3 = jnp.maximum(
        lax.dot_general(h2, w3_ref[...], (((1,), (0,)), ((), ())),
                        preferred_element_type=jnp.float32) + o3_ref[...],
        0.0)
    out_ref[0] = jnp.max(jnp.reshape(h3, (M_TILE, NSAMPLE, 128)), axis=1)


def _run_mlp(grows, centers, wg, w2t, w3t, o1, o2, o3):
    rows_tile = M_TILE * NSAMPLE
    return pl.pallas_call(
        _mlp_body,
        grid=(B, NPOINT // M_TILE),
        in_specs=[
            pl.BlockSpec((1, rows_tile, D_ROW), lambda b, t: (b, t, 0)),
            pl.BlockSpec((1, M_TILE, 3), lambda b, t: (b, t, 0)),
            pl.BlockSpec((3, 128), lambda b, t: (0, 0)),
            pl.BlockSpec((128, 128), lambda b, t: (0, 0)),
            pl.BlockSpec((128, 128), lambda b, t: (0, 0)),
            pl.BlockSpec((1, 128), lambda b, t: (0, 0)),
            pl.BlockSpec((1, 128), lambda b, t: (0, 0)),
            pl.BlockSpec((1, 128), lambda b, t: (0, 0)),
        ],
        out_specs=pl.BlockSpec((1, M_TILE, 128), lambda b, t: (b, t, 0)),
        out_shape=jax.ShapeDtypeStruct((B, NPOINT, 128), jnp.float32),
    )(grows, centers, wg, w2t, w3t, o1, o2, o3)


def kernel(xyz, features, seed_xyz, W1, g1, b1, m1, v1,
           W2, g2, b2, m2, v2, W3, g3, b3, m3, v3):
    xyz_t3 = jnp.transpose(xyz, (0, 2, 1))
    seed_t4 = jnp.reshape(jnp.transpose(seed_xyz, (0, 2, 1)), (B, 3, SUB, NL))

    s1 = g1 / jnp.sqrt(v1 + EPS)
    s2 = g2 / jnp.sqrt(v2 + EPS)
    s3 = g3 / jnp.sqrt(v3 + EPS)
    o1 = (b1 - m1 * s1)[None, :]
    o2 = (b2 - m2 * s2)[None, :]
    o3 = (b3 - m3 * s3)[None, :]
    w1f_t = jnp.transpose(W1[:, 3:] * s1[:, None])
    wg = jnp.transpose(W1[:, :3] * (s1 / RADIUS)[:, None])
    w2t = jnp.transpose(W2 * s2[:, None])
    w3t = jnp.transpose(W3 * s3[:, None])

    inds8 = _run_fps(seed_t4, seed_xyz)
    sample_inds = jnp.reshape(inds8, (B, NPOINT))
    inds3 = jnp.reshape(sample_inds, (B, NPOINT // C_TILE, C_TILE))

    idx, new_xyz = _run_ballq(xyz_t3, xyz, inds3)
    table = _run_table(features, xyz, w1f_t, wg)

    grows = _sc_gather(jnp.reshape(table, (B * N, D_ROW)),
                       jnp.reshape(idx, (B * NPOINT * NSAMPLE,)))
    out = _run_mlp(jnp.reshape(grows, (B, NPOINT * NSAMPLE, D_ROW)),
                   new_xyz, wg, w2t, w3t, o1, o2, o3)
    new_features = jnp.transpose(out, (0, 2, 1))
    return (new_xyz, new_features, sample_inds)


# speedup vs baseline: 2.2862x; 1.2068x over previous
import functools

import jax
import jax.numpy as jnp
import numpy as np
from jax import lax
from jax.experimental import pallas as pl
from jax.experimental.pallas import tpu as pltpu
from jax.experimental.pallas import tpu_sc as plsc

RADIUS = 0.3
NSAMPLE = 16
NPOINT = 512
EPS = 1e-5

B = 4
N = 16384
C = 256
SUB = 8
NL = N // SUB
D_ROW = 128
C_TILE = 32
M_TILE = 64


def _tree_min(x, stop=128):
    w = x.shape[1]
    while w > stop:
        w //= 2
        x = jnp.minimum(x[:, :w], x[:, w:])
    return jnp.min(x, axis=1, keepdims=True)


def _tree_max(x, stop=128):
    w = x.shape[1]
    while w > stop:
        w //= 2
        x = jnp.maximum(x[:, :w], x[:, w:])
    return jnp.max(x, axis=1, keepdims=True)


def _rmax(x):
    return jnp.max(_tree_max(x), axis=0, keepdims=True)


def _rmin(x):
    return jnp.min(_tree_min(x), axis=0, keepdims=True)


def _fps_body(seed_ref, rows_ref, inds_ref):
    sx = [seed_ref[b, 0] for b in range(B)]
    sy = [seed_ref[b, 1] for b in range(B)]
    sz = [seed_ref[b, 2] for b in range(B)]
    lin_f = ((lax.broadcasted_iota(jnp.int32, (SUB, NL), 0) * NL
              + lax.broadcasted_iota(jnp.int32, (SUB, NL), 1))
             .astype(jnp.float32))
    lin512 = (lax.broadcasted_iota(jnp.int32, (SUB, 64), 0) * 64
              + lax.broadcasted_iota(jnp.int32, (SUB, 64), 1))
    nf = jnp.float32(N)

    def body(i, state):
        dists, fs, inds = state
        onehot = lin512 == i
        new_dists, new_fs, new_inds = [], [], []
        for b in range(B):
            f = fs[b]
            ind_b = jnp.where(onehot, f, inds[b])
            row = rows_ref[b, pl.ds(f[0, 0], 1), :]
            dx = sx[b] - row[:, 0:1]
            dy = sy[b] - row[:, 1:2]
            dz = sz[b] - row[:, 2:3]
            d = (dx * dx + dy * dy) + dz * dz
            dist = jnp.minimum(dists[b], d)
            m = _rmax(dist)
            cand = jnp.where(dist == m, lin_f, nf)
            new_dists.append(dist)
            new_fs.append(_rmin(cand).astype(jnp.int32))
            new_inds.append(ind_b)
        return tuple(new_dists), tuple(new_fs), tuple(new_inds)

    init = (tuple(jnp.full((SUB, NL), 1e10, jnp.float32) for _ in range(B)),
            tuple(jnp.zeros((1, 1), jnp.int32) for _ in range(B)),
            tuple(jnp.zeros((SUB, 64), jnp.int32) for _ in range(B)))
    _, _, inds = lax.fori_loop(0, NPOINT, body, init)
    for b in range(B):
        inds_ref[b] = inds[b]


def _run_fps(seed_t4, seed_xyz):
    return pl.pallas_call(
        _fps_body,
        in_specs=[
            pl.BlockSpec((B, 3, SUB, NL), lambda: (0, 0, 0, 0)),
            pl.BlockSpec((B, N, 3), lambda: (0, 0, 0)),
        ],
        out_specs=pl.BlockSpec((B, SUB, 64), lambda: (0, 0, 0)),
        out_shape=jax.ShapeDtypeStruct((B, SUB, 64), jnp.int32),
    )(seed_t4, seed_xyz)


def _ballq_body(xyz_ref, rows_ref, inds_ref, idx_ref, nxyz_ref, ctr_scr):
    b = pl.program_id(0)
    x = xyz_ref[0, 0:1, :]
    y = xyz_ref[0, 1:2, :]
    z = xyz_ref[0, 2:3, :]
    t = pl.program_id(1)
    for j in range(C_TILE):
        s = inds_ref[0, t, j]
        ctr_scr[pl.ds(j, 1), :] = rows_ref[0, pl.ds(s, 1), :]
    c = ctr_scr[...]
    nxyz_ref[0] = c
    cx = c[:, 0:1]
    cy = c[:, 1:2]
    cz = c[:, 2:3]
    dx = cx - x
    dy = cy - y
    dz = cz - z
    d2 = (dx * dx + dy * dy) + dz * dz
    lin_f = lax.broadcasted_iota(jnp.int32, (C_TILE, N), 1).astype(jnp.float32)
    nf = jnp.float32(N)
    scores = jnp.where(d2 < np.float32(RADIUS * RADIUS), lin_f, nf)
    iota16 = lax.broadcasted_iota(jnp.int32, (1, NSAMPLE), 1)
    idxmat = jnp.zeros((C_TILE, NSAMPLE), jnp.int32)
    first = jnp.full((C_TILE, 1), N, jnp.int32)
    for k in range(NSAMPLE):
        m = _tree_min(scores)
        mi = m.astype(jnp.int32)
        if k == 0:
            first = mi
        slotval = jnp.where(mi < N, mi, first)
        scores = jnp.where(scores == m, nf, scores)
        idxmat = jnp.where(iota16 == k, slotval, idxmat)
    idx_ref[0] = idxmat + b * N


def _run_ballq(xyz_t3, xyz, inds3):
    return pl.pallas_call(
        _ballq_body,
        grid=(B, NPOINT // C_TILE),
        in_specs=[
            pl.BlockSpec((1, 3, N), lambda b, t: (b, 0, 0)),
            pl.BlockSpec((1, N, 3), lambda b, t: (b, 0, 0)),
            pl.BlockSpec((1, NPOINT // C_TILE, C_TILE), lambda b, t: (b, 0, 0),
                         memory_space=pltpu.SMEM),
        ],
        out_specs=[
            pl.BlockSpec((1, C_TILE, NSAMPLE), lambda b, t: (b, t, 0)),
            pl.BlockSpec((1, C_TILE, 3), lambda b, t: (b, t, 0)),
        ],
        out_shape=[
            jax.ShapeDtypeStruct((B, NPOINT, NSAMPLE), jnp.int32),
            jax.ShapeDtypeStruct((B, NPOINT, 3), jnp.float32),
        ],
        scratch_shapes=[pltpu.VMEM((C_TILE, 3), jnp.float32)],
    )(xyz_t3, xyz, inds3)


def _table_body(f_ref, xyz_ref, wt_ref, wg_ref, out_ref):
    f = f_ref[0]
    wt = wt_ref[...]
    out128 = lax.dot_general(f, wt, (((0,), (0,)), ((), ())),
                             preferred_element_type=jnp.float32)
    xyzb = xyz_ref[0]
    wg = wg_ref[...]
    xyzproj = (xyzb[:, 0:1] * wg[0:1, :] + xyzb[:, 1:2] * wg[1:2, :]
               + xyzb[:, 2:3] * wg[2:3, :])
    out_ref[0] = out128 + xyzproj


def _run_table(features, xyz, w1f_t, wg):
    n_tile = 512
    return pl.pallas_call(
        _table_body,
        grid=(B, N // n_tile),
        in_specs=[
            pl.BlockSpec((1, C, n_tile), lambda b, t: (b, 0, t)),
            pl.BlockSpec((1, n_tile, 3), lambda b, t: (b, t, 0)),
            pl.BlockSpec((C, 128), lambda b, t: (0, 0)),
            pl.BlockSpec((3, 128), lambda b, t: (0, 0)),
        ],
        out_specs=pl.BlockSpec((1, n_tile, D_ROW), lambda b, t: (b, t, 0)),
        out_shape=jax.ShapeDtypeStruct((B, N, D_ROW), jnp.float32),
    )(features, xyz, w1f_t, wg)


_SC_CHUNK = 128


def _sc_gather(table, idx):
    tot = idx.shape[0]
    info = plsc.get_sparse_core_info()
    nw = info.num_cores * info.num_subcores
    per_w = tot // nw
    n_chunk = per_w // _SC_CHUNK
    mesh = plsc.VectorSubcoreMesh(core_axis_name="c", subcore_axis_name="s")

    @functools.partial(
        pl.kernel,
        mesh=mesh,
        out_type=jax.ShapeDtypeStruct((tot, D_ROW), jnp.float32),
        scratch_types=[
            pltpu.VMEM((_SC_CHUNK,), jnp.int32),
            pltpu.VMEM((_SC_CHUNK, D_ROW), jnp.float32),
            pltpu.SemaphoreType.DMA,
        ],
    )
    def k(tab_hbm, idx_hbm, out_hbm, idx_v, rows_v, sem):
        wid = lax.axis_index("s") * info.num_cores + lax.axis_index("c")
        base = wid * per_w

        def body(ci, carry):
            off = base + ci * _SC_CHUNK
            pltpu.sync_copy(idx_hbm.at[pl.ds(off, _SC_CHUNK)], idx_v)
            pltpu.async_copy(tab_hbm.at[idx_v], rows_v, sem).wait()
            pltpu.sync_copy(rows_v, out_hbm.at[pl.ds(off, _SC_CHUNK)])
            return carry

        lax.fori_loop(0, n_chunk, body, 0)

    return k(table, idx)


def _mlp_body(g_ref, ctr_ref, wg_ref, w2_ref, w3_ref,
              o1_ref, o2_ref, o3_ref, out_ref):
    h1p = g_ref[0]
    c = ctr_ref[0]
    wg = wg_ref[...]
    cc = (c[:, 0:1] * wg[0:1, :] + c[:, 1:2] * wg[1:2, :]
          + c[:, 2:3] * wg[2:3, :])
    ccrep = jnp.reshape(
        jnp.broadcast_to(cc[:, None, :], (M_TILE, NSAMPLE, 128)),
        (M_TILE * NSAMPLE, 128))
    h1 = jnp.maximum(h1p - ccrep + o1_ref[...], 0.0)
    h2 = jnp.maximum(
        lax.dot_general(h1, w2_ref[...], (((1,), (0,)), ((), ())),
                        preferred_element_type=jnp.float32) + o2_ref[...],
        0.0)
    h3 = jnp.maximum(
        lax.dot_general(h2, w3_ref[...], (((1,), (0,)), ((), ())),
                        preferred_element_type=jnp.float32) + o3_ref[...],
        0.0)
    out_ref[0] = jnp.max(jnp.reshape(h3, (M_TILE, NSAMPLE, 128)), axis=1)


def _run_mlp(grows, centers, wg, w2t, w3t, o1, o2, o3):
    rows_tile = M_TILE * NSAMPLE
    return pl.pallas_call(
        _mlp_body,
        grid=(B, NPOINT // M_TILE),
        in_specs=[
            pl.BlockSpec((1, rows_tile, D_ROW), lambda b, t: (b, t, 0)),
            pl.BlockSpec((1, M_TILE, 3), lambda b, t: (b, t, 0)),
            pl.BlockSpec((3, 128), lambda b, t: (0, 0)),
            pl.BlockSpec((128, 128), lambda b, t: (0, 0)),
            pl.BlockSpec((128, 128), lambda b, t: (0, 0)),
            pl.BlockSpec((1, 128), lambda b, t: (0, 0)),
            pl.BlockSpec((1, 128), lambda b, t: (0, 0)),
            pl.BlockSpec((1, 128), lambda b, t: (0, 0)),
        ],
        out_specs=pl.BlockSpec((1, M_TILE, 128), lambda b, t: (b, t, 0)),
        out_shape=jax.ShapeDtypeStruct((B, NPOINT, 128), jnp.float32),
    )(grows, centers, wg, w2t, w3t, o1, o2, o3)


def kernel(xyz, features, seed_xyz, W1, g1, b1, m1, v1,
           W2, g2, b2, m2, v2, W3, g3, b3, m3, v3):
    xyz_t3 = jnp.transpose(xyz, (0, 2, 1))
    seed_t4 = jnp.reshape(jnp.transpose(seed_xyz, (0, 2, 1)), (B, 3, SUB, NL))

    s1 = g1 / jnp.sqrt(v1 + EPS)
    s2 = g2 / jnp.sqrt(v2 + EPS)
    s3 = g3 / jnp.sqrt(v3 + EPS)
    o1 = (b1 - m1 * s1)[None, :]
    o2 = (b2 - m2 * s2)[None, :]
    o3 = (b3 - m3 * s3)[None, :]
    w1f_t = jnp.transpose(W1[:, 3:] * s1[:, None])
    wg = jnp.transpose(W1[:, :3] * (s1 / RADIUS)[:, None])
    w2t = jnp.transpose(W2 * s2[:, None])
    w3t = jnp.transpose(W3 * s3[:, None])

    inds8 = _run_fps(seed_t4, seed_xyz)
    sample_inds = jnp.reshape(inds8, (B, NPOINT))
    inds3 = jnp.reshape(sample_inds, (B, NPOINT // C_TILE, C_TILE))

    idx, new_xyz = _run_ballq(xyz_t3, xyz, inds3)
    table = _run_table(features, xyz, w1f_t, wg)

    grows = _sc_gather(jnp.reshape(table, (B * N, D_ROW)),
                       jnp.reshape(idx, (B * NPOINT * NSAMPLE,)))
    out = _run_mlp(jnp.reshape(grows, (B, NPOINT * NSAMPLE, D_ROW)),
                   new_xyz, wg, w2t, w3t, o1, o2, o3)
    new_features = jnp.transpose(out, (0, 2, 1))
    return (new_xyz, new_features, sample_inds)


# speedup vs baseline: 2.3114x; 1.0110x over previous
import functools

import jax
import jax.numpy as jnp
import numpy as np
from jax import lax
from jax.experimental import pallas as pl
from jax.experimental.pallas import tpu as pltpu
from jax.experimental.pallas import tpu_sc as plsc

RADIUS = 0.3
NSAMPLE = 16
NPOINT = 512
EPS = 1e-5

B = 4
N = 16384
C = 256
SUB = 8
NL = N // SUB
D_ROW = 128
C_TILE = 32
M_TILE = 128


def _tree_min(x, stop=128):
    w = x.shape[1]
    while w > stop:
        w //= 2
        x = jnp.minimum(x[:, :w], x[:, w:])
    return jnp.min(x, axis=1, keepdims=True)


def _tree_max(x, stop=128):
    w = x.shape[1]
    while w > stop:
        w //= 2
        x = jnp.maximum(x[:, :w], x[:, w:])
    return jnp.max(x, axis=1, keepdims=True)


def _rmax(x):
    return jnp.max(_tree_max(x), axis=0, keepdims=True)


def _rmin(x):
    return jnp.min(_tree_min(x), axis=0, keepdims=True)


def _fps_body(seed_ref, rows_ref, inds_ref):
    sx = [seed_ref[b, 0] for b in range(B)]
    sy = [seed_ref[b, 1] for b in range(B)]
    sz = [seed_ref[b, 2] for b in range(B)]
    lin_f = ((lax.broadcasted_iota(jnp.int32, (SUB, NL), 0) * NL
              + lax.broadcasted_iota(jnp.int32, (SUB, NL), 1))
             .astype(jnp.float32))
    lin512 = (lax.broadcasted_iota(jnp.int32, (SUB, 64), 0) * 64
              + lax.broadcasted_iota(jnp.int32, (SUB, 64), 1))
    nf = jnp.float32(N)

    def body(i, state):
        dists, fs, inds = state
        onehot = lin512 == i
        new_dists, new_fs, new_inds = [], [], []
        for b in range(B):
            f = fs[b]
            ind_b = jnp.where(onehot, f, inds[b])
            row = rows_ref[b, pl.ds(f[0, 0], 1), :]
            dx = sx[b] - row[:, 0:1]
            dy = sy[b] - row[:, 1:2]
            dz = sz[b] - row[:, 2:3]
            d = (dx * dx + dy * dy) + dz * dz
            dist = jnp.minimum(dists[b], d)
            m = _rmax(dist)
            cand = jnp.where(dist == m, lin_f, nf)
            new_dists.append(dist)
            new_fs.append(_rmin(cand).astype(jnp.int32))
            new_inds.append(ind_b)
        return tuple(new_dists), tuple(new_fs), tuple(new_inds)

    init = (tuple(jnp.full((SUB, NL), 1e10, jnp.float32) for _ in range(B)),
            tuple(jnp.zeros((1, 1), jnp.int32) for _ in range(B)),
            tuple(jnp.zeros((SUB, 64), jnp.int32) for _ in range(B)))
    _, _, inds = lax.fori_loop(0, NPOINT, body, init)
    for b in range(B):
        inds_ref[b] = inds[b]


def _run_fps(seed_t4, seed_xyz):
    return pl.pallas_call(
        _fps_body,
        in_specs=[
            pl.BlockSpec((B, 3, SUB, NL), lambda: (0, 0, 0, 0)),
            pl.BlockSpec((B, N, 3), lambda: (0, 0, 0)),
        ],
        out_specs=pl.BlockSpec((B, SUB, 64), lambda: (0, 0, 0)),
        out_shape=jax.ShapeDtypeStruct((B, SUB, 64), jnp.int32),
    )(seed_t4, seed_xyz)


def _ballq_body(xyz_ref, rows_ref, inds_ref, idx_ref, nxyz_ref, ctr_scr):
    b = pl.program_id(0)
    x = xyz_ref[0, 0:1, :]
    y = xyz_ref[0, 1:2, :]
    z = xyz_ref[0, 2:3, :]
    t = pl.program_id(1)
    for j in range(C_TILE):
        s = inds_ref[0, t, j]
        ctr_scr[pl.ds(j, 1), :] = rows_ref[0, pl.ds(s, 1), :]
    c = ctr_scr[...]
    nxyz_ref[0] = c
    cx = c[:, 0:1]
    cy = c[:, 1:2]
    cz = c[:, 2:3]
    dx = cx - x
    dy = cy - y
    dz = cz - z
    d2 = (dx * dx + dy * dy) + dz * dz
    lin_f = lax.broadcasted_iota(jnp.int32, (C_TILE, N), 1).astype(jnp.float32)
    nf = jnp.float32(N)
    scores = jnp.where(d2 < np.float32(RADIUS * RADIUS), lin_f, nf)
    iota16 = lax.broadcasted_iota(jnp.int32, (1, NSAMPLE), 1)
    idxmat = jnp.zeros((C_TILE, NSAMPLE), jnp.int32)
    first = jnp.full((C_TILE, 1), N, jnp.int32)
    for k in range(NSAMPLE):
        m = _tree_min(scores)
        mi = m.astype(jnp.int32)
        if k == 0:
            first = mi
        slotval = jnp.where(mi < N, mi, first)
        scores = jnp.where(scores == m, nf, scores)
        idxmat = jnp.where(iota16 == k, slotval, idxmat)
    idx_ref[0] = idxmat + b * N


def _run_ballq(xyz_t3, xyz, inds3):
    return pl.pallas_call(
        _ballq_body,
        grid=(B, NPOINT // C_TILE),
        in_specs=[
            pl.BlockSpec((1, 3, N), lambda b, t: (b, 0, 0)),
            pl.BlockSpec((1, N, 3), lambda b, t: (b, 0, 0)),
            pl.BlockSpec((1, NPOINT // C_TILE, C_TILE), lambda b, t: (b, 0, 0),
                         memory_space=pltpu.SMEM),
        ],
        out_specs=[
            pl.BlockSpec((1, C_TILE, NSAMPLE), lambda b, t: (b, t, 0)),
            pl.BlockSpec((1, C_TILE, 3), lambda b, t: (b, t, 0)),
        ],
        out_shape=[
            jax.ShapeDtypeStruct((B, NPOINT, NSAMPLE), jnp.int32),
            jax.ShapeDtypeStruct((B, NPOINT, 3), jnp.float32),
        ],
        scratch_shapes=[pltpu.VMEM((C_TILE, 3), jnp.float32)],
    )(xyz_t3, xyz, inds3)


def _table_body(f_ref, xyz_ref, wt_ref, wg_ref, out_ref):
    f = f_ref[0]
    wt = wt_ref[...]
    out128 = lax.dot_general(f, wt, (((0,), (0,)), ((), ())),
                             preferred_element_type=jnp.float32)
    xyzb = xyz_ref[0]
    wg = wg_ref[...]
    xyzproj = (xyzb[:, 0:1] * wg[0:1, :] + xyzb[:, 1:2] * wg[1:2, :]
               + xyzb[:, 2:3] * wg[2:3, :])
    out_ref[0] = out128 + xyzproj


def _run_table(features, xyz, w1f_t, wg):
    n_tile = 512
    return pl.pallas_call(
        _table_body,
        grid=(B, N // n_tile),
        in_specs=[
            pl.BlockSpec((1, C, n_tile), lambda b, t: (b, 0, t)),
            pl.BlockSpec((1, n_tile, 3), lambda b, t: (b, t, 0)),
            pl.BlockSpec((C, 128), lambda b, t: (0, 0)),
            pl.BlockSpec((3, 128), lambda b, t: (0, 0)),
        ],
        out_specs=pl.BlockSpec((1, n_tile, D_ROW), lambda b, t: (b, t, 0)),
        out_shape=jax.ShapeDtypeStruct((B, N, D_ROW), jnp.float32),
    )(features, xyz, w1f_t, wg)


_SC_CHUNK = 128


def _sc_gather(table, idx):
    tot = idx.shape[0]
    info = plsc.get_sparse_core_info()
    nw = info.num_cores * info.num_subcores
    per_w = tot // nw
    n_chunk = per_w // _SC_CHUNK
    mesh = plsc.VectorSubcoreMesh(core_axis_name="c", subcore_axis_name="s")

    @functools.partial(
        pl.kernel,
        mesh=mesh,
        out_type=jax.ShapeDtypeStruct((tot, D_ROW), jnp.float32),
        scratch_types=[
            pltpu.VMEM((_SC_CHUNK,), jnp.int32),
            pltpu.VMEM((_SC_CHUNK, D_ROW), jnp.float32),
            pltpu.SemaphoreType.DMA,
        ],
    )
    def k(tab_hbm, idx_hbm, out_hbm, idx_v, rows_v, sem):
        wid = lax.axis_index("s") * info.num_cores + lax.axis_index("c")
        base = wid * per_w

        def body(ci, carry):
            off = base + ci * _SC_CHUNK
            pltpu.sync_copy(idx_hbm.at[pl.ds(off, _SC_CHUNK)], idx_v)
            pltpu.async_copy(tab_hbm.at[idx_v], rows_v, sem).wait()
            pltpu.sync_copy(rows_v, out_hbm.at[pl.ds(off, _SC_CHUNK)])
            return carry

        lax.fori_loop(0, n_chunk, body, 0)

    return k(table, idx)


def _mlp_body(g_ref, ctr_ref, wg_ref, w2_ref, w3_ref,
              o1_ref, o2_ref, o3_ref, out_ref):
    h1p = g_ref[0]
    c = ctr_ref[0]
    wg = wg_ref[...]
    cc = (c[:, 0:1] * wg[0:1, :] + c[:, 1:2] * wg[1:2, :]
          + c[:, 2:3] * wg[2:3, :])
    ccrep = jnp.reshape(
        jnp.broadcast_to(cc[:, None, :], (M_TILE, NSAMPLE, 128)),
        (M_TILE * NSAMPLE, 128))
    h1 = jnp.maximum(h1p - ccrep + o1_ref[...], 0.0)
    h2 = jnp.maximum(
        lax.dot_general(h1, w2_ref[...], (((1,), (0,)), ((), ())),
                        preferred_element_type=jnp.float32) + o2_ref[...],
        0.0)
    h3 = jnp.maximum(
        lax.dot_general(h2, w3_ref[...], (((1,), (0,)), ((), ())),
                        preferred_element_type=jnp.float32) + o3_ref[...],
        0.0)
    y = jnp.max(jnp.reshape(h3, (M_TILE, NSAMPLE, 128)), axis=1)
    out_ref[0] = jnp.transpose(y)


def _run_mlp(grows, centers, wg, w2t, w3t, o1, o2, o3):
    rows_tile = M_TILE * NSAMPLE
    return pl.pallas_call(
        _mlp_body,
        grid=(B, NPOINT // M_TILE),
        in_specs=[
            pl.BlockSpec((1, rows_tile, D_ROW), lambda b, t: (b, t, 0)),
            pl.BlockSpec((1, M_TILE, 3), lambda b, t: (b, t, 0)),
            pl.BlockSpec((3, 128), lambda b, t: (0, 0)),
            pl.BlockSpec((128, 128), lambda b, t: (0, 0)),
            pl.BlockSpec((128, 128), lambda b, t: (0, 0)),
            pl.BlockSpec((1, 128), lambda b, t: (0, 0)),
            pl.BlockSpec((1, 128), lambda b, t: (0, 0)),
            pl.BlockSpec((1, 128), lambda b, t: (0, 0)),
        ],
        out_specs=pl.BlockSpec((1, 128, M_TILE), lambda b, t: (b, 0, t)),
        out_shape=jax.ShapeDtypeStruct((B, 128, NPOINT), jnp.float32),
    )(grows, centers, wg, w2t, w3t, o1, o2, o3)


def kernel(xyz, features, seed_xyz, W1, g1, b1, m1, v1,
           W2, g2, b2, m2, v2, W3, g3, b3, m3, v3):
    xyz_t3 = jnp.transpose(xyz, (0, 2, 1))
    seed_t4 = jnp.reshape(jnp.transpose(seed_xyz, (0, 2, 1)), (B, 3, SUB, NL))

    s1 = g1 / jnp.sqrt(v1 + EPS)
    s2 = g2 / jnp.sqrt(v2 + EPS)
    s3 = g3 / jnp.sqrt(v3 + EPS)
    o1 = (b1 - m1 * s1)[None, :]
    o2 = (b2 - m2 * s2)[None, :]
    o3 = (b3 - m3 * s3)[None, :]
    w1f_t = jnp.transpose(W1[:, 3:] * s1[:, None])
    wg = jnp.transpose(W1[:, :3] * (s1 / RADIUS)[:, None])
    w2t = jnp.transpose(W2 * s2[:, None])
    w3t = jnp.transpose(W3 * s3[:, None])

    inds8 = _run_fps(seed_t4, seed_xyz)
    sample_inds = jnp.reshape(inds8, (B, NPOINT))
    inds3 = jnp.reshape(sample_inds, (B, NPOINT // C_TILE, C_TILE))

    idx, new_xyz = _run_ballq(xyz_t3, xyz, inds3)
    table = _run_table(features, xyz, w1f_t, wg)

    grows = _sc_gather(jnp.reshape(table, (B * N, D_ROW)),
                       jnp.reshape(idx, (B * NPOINT * NSAMPLE,)))
    new_features = _run_mlp(jnp.reshape(grows, (B, NPOINT * NSAMPLE, D_ROW)),
                            new_xyz, wg, w2t, w3t, o1, o2, o3)
    return (new_xyz, new_features, sample_inds)
